# Initial kernel scaffold; baseline (speedup 1.0000x reference)
import jax
import jax.numpy as jnp
from jax.experimental import pallas as pl


def kernel(x, edge_index, W_enc, b_enc, W1, b1, W2, b2, W3, b3, W_dec, b_dec):
    raise NotImplementedError("write your pallas kernel here")


# speedup vs baseline: 7.3142x; 7.3142x over previous
import functools

import jax
import jax.numpy as jnp
from jax import lax
from jax.experimental import pallas as pl
from jax.experimental.pallas import tpu as pltpu
from jax.experimental.pallas import tpu_sc as plsc

N = 10000
E = 320000
DH = 64
NC = 2
NS = 16
NW = NC * NS
EPW = E // NW
KCH = 80
NCHUNK = EPW // KCH
RPT = 624
TAIL0 = RPT * NS
TAILN = N - TAIL0

ROW_BLK = 1000
GRID = N // ROW_BLK


def _make_prop(d):
    mesh = plsc.VectorSubcoreMesh(
        core_axis_name="c", subcore_axis_name="s", num_cores=NC, num_subcores=NS
    )

    @functools.partial(
        pl.kernel,
        out_type=jax.ShapeDtypeStruct((NC * N, d), jnp.float32),
        mesh=mesh,
        scratch_types=[
            pltpu.VMEM_SHARED((N, d), jnp.float32),
            pltpu.VMEM((KCH,), jnp.int32),
            pltpu.VMEM((KCH,), jnp.int32),
            pltpu.VMEM((KCH, d), jnp.float32),
            pltpu.SemaphoreType.DMA,
        ],
    )
    def prop(table, idxg, idxs, zeros, out, acc, gidx_v, sidx_v, rows_v, sem):
        cid = lax.axis_index("c")
        sid = lax.axis_index("s")
        wid = cid * NS + sid
        r0 = sid * RPT
        pltpu.sync_copy(zeros.at[pl.ds(r0, RPT)], acc.at[pl.ds(r0, RPT)])

        @pl.when(sid == NS - 1)
        def _():
            pltpu.sync_copy(zeros.at[pl.ds(TAIL0, TAILN)], acc.at[pl.ds(TAIL0, TAILN)])

        plsc.subcore_barrier()
        base = wid * EPW

        def body(i, carry):
            off = pl.multiple_of(base + i * KCH, 8)
            pltpu.sync_copy(idxg.at[pl.ds(off, KCH)], gidx_v)
            pltpu.sync_copy(idxs.at[pl.ds(off, KCH)], sidx_v)
            pltpu.async_copy(table.at[gidx_v], rows_v, sem).wait()
            pltpu.sync_copy(rows_v, acc.at[sidx_v], add=True)
            return carry

        lax.fori_loop(0, NCHUNK, body, 0)
        plsc.subcore_barrier()
        pltpu.sync_copy(
            acc.at[pl.ds(r0, RPT)], out.at[pl.ds(cid * N + r0, RPT)]
        )

        @pl.when(sid == NS - 1)
        def _():
            pltpu.sync_copy(
                acc.at[pl.ds(TAIL0, TAILN)], out.at[pl.ds(cid * N + TAIL0, TAILN)]
            )

    return prop


_prop128 = _make_prop(128)


def _row_spec(cols):
    return pl.BlockSpec((ROW_BLK, cols), lambda i: (i, 0))


def _pair_spec(cols):
    return pl.BlockSpec((ROW_BLK, cols), lambda i: (i, 0))


def _full_spec(rows, cols):
    return pl.BlockSpec((rows, cols), lambda i: (0, 0))


def _tc_call(body, in_specs, out_shape, out_specs):
    return pl.pallas_call(
        body,
        grid=(GRID,),
        in_specs=in_specs,
        out_shape=out_shape,
        out_specs=out_specs,
    )


def _enc_body(x_ref, w_ref, b_ref, y_ref):
    y = jnp.dot(x_ref[...], w_ref[...], preferred_element_type=jnp.float32)
    y_ref[...] = jnp.maximum(y + b_ref[...], 0.0)


def _stage1_body(xr, yr, w1r, dvr, ur):
    acc = jnp.dot(xr[...], w1r[:DH], preferred_element_type=jnp.float32)
    acc += jnp.dot(yr[...], w1r[DH:], preferred_element_type=jnp.float32)
    ur[...] = dvr[...] * acc


def _stage2_body(pa, pb, ur, dvr, b1r, w2pr, o1r, u1r):
    o1 = jnp.tanh(dvr[...] * (pa[...] + pb[...] + ur[...]) + b1r[...])
    o1r[...] = o1
    u1r[...] = dvr[...] * jnp.dot(o1, w2pr[...], preferred_element_type=jnp.float32)


def _stage3_body(pa, pb, u1r, dvr, b2r, cr, w3pr, v2r):
    o2 = jnp.tanh(dvr[...] * (pa[...] + pb[...] + u1r[...]) + b2r[...])
    v2r[...] = dvr[...] * (1.0 - o2 * o2) * (cr[...] * w3pr[...])


def _stage4_body(qa, qb, v2r, dvr, o1r, w2pr, v1r):
    t = dvr[...] * (qa[...] + qb[...] + v2r[...])
    go1 = jnp.dot(t, w2pr[...].T, preferred_element_type=jnp.float32)
    o1 = o1r[...]
    v1r[...] = dvr[...] * (1.0 - o1 * o1) * go1


def _stage5_body(qa, qb, v1r, dvr, w1r, xr, yr, xnr, ynr):
    z = dvr[...] * (qa[...] + qb[...] + v1r[...])
    xnr[...] = xr[...] + jnp.dot(z, w1r[DH:].T, preferred_element_type=jnp.float32)
    ynr[...] = yr[...] - jnp.dot(z, w1r[:DH].T, preferred_element_type=jnp.float32)


def _dec_body(xr, wr, br, outr):
    outr[...] = jnp.dot(xr[...], wr[...], preferred_element_type=jnp.float32) + br[...]


def kernel(x, edge_index, W_enc, b_enc, W1, b1, W2, b2, W3, b3, W_dec, b_dec):
    f32 = jnp.float32
    src = edge_index[0]
    dst = edge_index[1]
    z128 = jnp.zeros((N, 128), f32)
    ones128 = jnp.ones((N, 128), f32)

    degp = _prop128(ones128, src, dst, z128)
    deg = degp[:N, 0] + degp[N:, 0] + 1.0
    dinv = lax.rsqrt(deg)
    dinv128 = jnp.broadcast_to(dinv[:, None], (N, 128))
    ctp = _prop128(dinv128, dst, src, z128)
    c0 = dinv * (ctp[:N, 0] + ctp[N:, 0] + dinv)

    dv2 = dinv[:, None]
    c2 = c0[:, None]
    b1_ = b1[None, :]
    b2p = jnp.concatenate([b2, jnp.zeros((DH,), f32)])[None, :]
    benc_ = b_enc[None, :]
    bdec_ = b_dec[None, :]
    W2p = jnp.concatenate([W2, jnp.zeros((128, DH), f32)], axis=1)
    w3p = jnp.concatenate([W3[:, 0], jnp.zeros((DH,), f32)])[None, :]

    sc_dv = pl.BlockSpec((ROW_BLK, 1), lambda i: (i, 0))

    Y = _tc_call(
        _enc_body,
        [_row_spec(128), _full_spec(128, DH), _full_spec(1, DH)],
        jax.ShapeDtypeStruct((N, DH), f32),
        _row_spec(DH),
    )(x, W_enc, benc_)
    X = Y

    stage1 = _tc_call(
        _stage1_body,
        [_row_spec(DH), _row_spec(DH), _full_spec(128, 128), sc_dv],
        jax.ShapeDtypeStruct((N, 128), f32),
        _row_spec(128),
    )
    stage2 = _tc_call(
        _stage2_body,
        [_row_spec(128), _row_spec(128), _row_spec(128), sc_dv,
         _full_spec(1, 128), _full_spec(128, 128)],
        [jax.ShapeDtypeStruct((N, 128), f32), jax.ShapeDtypeStruct((N, 128), f32)],
        [_row_spec(128), _row_spec(128)],
    )
    stage3 = _tc_call(
        _stage3_body,
        [_row_spec(128), _row_spec(128), _row_spec(128), sc_dv,
         _full_spec(1, 128), sc_dv, _full_spec(1, 128)],
        jax.ShapeDtypeStruct((N, 128), f32),
        _row_spec(128),
    )
    stage4 = _tc_call(
        _stage4_body,
        [_row_spec(128), _row_spec(128), _row_spec(128), sc_dv,
         _row_spec(128), _full_spec(128, 128)],
        jax.ShapeDtypeStruct((N, 128), f32),
        _row_spec(128),
    )
    stage5 = _tc_call(
        _stage5_body,
        [_row_spec(128), _row_spec(128), _row_spec(128), sc_dv,
         _full_spec(128, 128), _row_spec(DH), _row_spec(DH)],
        [jax.ShapeDtypeStruct((N, DH), f32), jax.ShapeDtypeStruct((N, DH), f32)],
        [_row_spec(DH), _row_spec(DH)],
    )

    for _ in range(2):
        u0 = stage1(X, Y, W1, dv2)
        p0 = _prop128(u0, src, dst, z128)
        o1, u1 = stage2(p0[:N], p0[N:], u0, dv2, b1_, W2p)
        p1 = _prop128(u1, src, dst, z128)
        v2 = stage3(p1[:N], p1[N:], u1, dv2, b2p, c2, w3p)
        q2 = _prop128(v2, dst, src, z128)
        v1 = stage4(q2[:N], q2[N:], v2, dv2, o1, W2p)
        q1 = _prop128(v1, dst, src, z128)
        X, Y = stage5(q1[:N], q1[N:], v1, dv2, W1, X, Y)

    out = _tc_call(
        _dec_body,
        [_row_spec(DH), _full_spec(DH, 16), _full_spec(1, 16)],
        jax.ShapeDtypeStruct((N, 16), f32),
        _row_spec(16),
    )(X, W_dec, bdec_)
    return out
